# baseline (speedup 1.0000x reference)
import jax
import jax.numpy as jnp
from jax.experimental import pallas as pl
from jax.experimental.pallas import tpu as pltpu
from jax.experimental.pallas import tpu_sc as plsc

B = 16
S = 512
IN = 128
H = 128
G = 128
LH = 256
HALF = 64
BS = B * S
CH = 4
CB = B // CH

_BF = jnp.bfloat16
_F32 = jnp.float32


_NC, _NS = 2, 16
_NW = _NC * _NS
_B_PER_W = BS // _NW


def _row_gather_body(table_hbm, idx_hbm, out_hbm, idx_v, rows_v, sem):
    wid = jax.lax.axis_index("s") * _NC + jax.lax.axis_index("c")
    base = wid * _B_PER_W
    pltpu.sync_copy(idx_hbm.at[pl.ds(base, _B_PER_W)], idx_v)
    pltpu.async_copy(table_hbm.at[idx_v], rows_v, sem).wait()
    pltpu.sync_copy(rows_v, out_hbm.at[pl.ds(base, _B_PER_W)])


def _row_gather(table, idx_flat):
    k = pl.kernel(
        _row_gather_body,
        out_type=jax.ShapeDtypeStruct((BS, table.shape[1]), _F32),
        mesh=plsc.VectorSubcoreMesh(core_axis_name="c", subcore_axis_name="s"),
        scratch_types=[
            pltpu.VMEM((_B_PER_W,), jnp.int32),
            pltpu.VMEM((_B_PER_W, table.shape[1]), _F32),
            pltpu.SemaphoreType.DMA,
        ],
    )
    return k(table, idx_flat)


def _perm_tm_to_bm():
    r = jnp.arange(BS, dtype=jnp.int32)
    return (r % S) * B + r // S


def _perm_bm_to_tm():
    r = jnp.arange(BS, dtype=jnp.int32)
    return (r % B) * S + r // B


def _ctx_lstm_kernel(x_ref, wih_ref, whh_ref, b_ref, out_ref, xp_ref):
    def proj(k, _):
        rows = pl.ds(k * 256, 256)
        xp_ref[rows, :] = jnp.dot(x_ref[rows, :].astype(_BF), wih_ref[...],
                                  preferred_element_type=_F32) + b_ref[...]
        return _
    jax.lax.fori_loop(0, BS // 256, proj, 0)

    def step(t, carry):
        xp_t = xp_ref[pl.ds(t * B, B), :]
        new = []
        for g in range(CH):
            h, c = carry[2 * g], carry[2 * g + 1]
            gates = xp_t[g * CB:(g + 1) * CB, :] + jnp.dot(
                h.astype(_BF), whh_ref[...], preferred_element_type=_F32)
            sig = jax.nn.sigmoid(gates[:, 0:3 * H])
            gg = jnp.tanh(gates[:, 3 * H:4 * H])
            c = sig[:, H:2 * H] * c + sig[:, 0:H] * gg
            h = sig[:, 2 * H:3 * H] * jnp.tanh(c)
            out_ref[t, pl.ds(g * CB, CB), :] = h
            new += [h, c]
        return tuple(new)

    z = jnp.zeros((CB, H), _F32)
    jax.lax.fori_loop(0, S, step, (z, z) * CH)


def _ctx_lstm(x_flat, wih_t, whh_t, bias):
    return pl.pallas_call(
        _ctx_lstm_kernel,
        out_shape=jax.ShapeDtypeStruct((S, B, H), _F32),
        scratch_shapes=[pltpu.VMEM((BS, 4 * H), _F32)],
    )(x_flat, wih_t, whh_t, bias)


def _gcn_kernel(adj_ref, h_ref, wg_ref, bg_ref, out_ref):
    adj = adj_ref[0]
    rs = jnp.sum(adj, axis=1, keepdims=True) + 1e-8
    m = jnp.dot(adj.astype(_BF), h_ref[0].astype(_BF),
                preferred_element_type=_F32)
    m = m / rs
    g = jnp.dot(m.astype(_BF), wg_ref[...], preferred_element_type=_F32)
    out_ref[0] = jnp.maximum(g + bg_ref[...], 0.0)


def _gcn(adj, h_bm, wg_t, bg):
    return pl.pallas_call(
        _gcn_kernel,
        grid=(B,),
        in_specs=[
            pl.BlockSpec((1, S, S), lambda b: (b, 0, 0)),
            pl.BlockSpec((1, S, H), lambda b: (b, 0, 0)),
            pl.BlockSpec((H, G), lambda b: (0, 0)),
            pl.BlockSpec((1, G), lambda b: (0, 0)),
        ],
        out_specs=pl.BlockSpec((1, S, G), lambda b: (b, 0, 0)),
        out_shape=jax.ShapeDtypeStruct((B, S, G), _F32),
    )(adj, h_bm, wg_t, bg)


def _bilstm_kernel(x_ref, wxf_ref, wxb_ref, whh_ref, b_ref, out_ref,
                   xpf_ref, xpb_ref):
    def proj(k, _):
        rows = pl.ds(k * 256, 256)
        xb = x_ref[rows, :].astype(_BF)
        xpf_ref[rows, :] = (jnp.dot(xb, wxf_ref[...],
                                    preferred_element_type=_F32)
                            + b_ref[...]).astype(_BF)
        xpb_ref[rows, :] = jnp.dot(xb, wxb_ref[...],
                                   preferred_element_type=_F32).astype(_BF)
        return _
    jax.lax.fori_loop(0, BS // 256, proj, 0)

    def step(t, carry):
        tb = S - 1 - t
        xpf_t = xpf_ref[pl.ds(t * B, B), :].astype(_F32)
        xpb_t = xpb_ref[pl.ds(tb * B, B), :].astype(_F32)
        new = []
        for g in range(CH):
            h, c = carry[2 * g], carry[2 * g + 1]
            rows = slice(g * CB, (g + 1) * CB)
            gates = xpf_t[rows, :] + xpb_t[rows, :] + jnp.dot(
                h.astype(_BF), whh_ref[...], preferred_element_type=_F32)
            sig = jax.nn.sigmoid(gates[:, 0:3 * H])
            gg = jnp.tanh(gates[:, 3 * H:4 * H])
            c = sig[:, H:2 * H] * c + sig[:, 0:H] * gg
            h = sig[:, 2 * H:3 * H] * jnp.tanh(c)
            out_ref[t, pl.ds(g * CB, CB), 0:HALF] = h[:, 0:HALF]
            out_ref[tb, pl.ds(g * CB, CB), HALF:H] = h[:, HALF:H]
            new += [h, c]
        return tuple(new)

    z = jnp.zeros((CB, H), _F32)
    jax.lax.fori_loop(0, S, step, (z, z) * CH)


def _bilstm(x_flat, wxf, wxb, whh_bd, bias_cat):
    return pl.pallas_call(
        _bilstm_kernel,
        out_shape=jax.ShapeDtypeStruct((S, B, H), _F32),
        scratch_shapes=[
            pltpu.VMEM((BS, 4 * H), _BF),
            pltpu.VMEM((BS, 4 * H), _BF),
        ],
    )(x_flat, wxf, wxb, whh_bd, bias_cat)


def _pack_bilstm(fW, fU, fb1, fb2, bW, bU, bb1, bb2):
    perm = jnp.array([0, 1, 3, 2])

    def cols(Wt, n_in):
        return Wt.reshape(n_in, 4, HALF)[:, perm, :]

    wxf = jnp.zeros((G, 4, 2, HALF), _F32)
    wxf = wxf.at[:, :, 0, :].set(cols(fW.T, G))
    wxb = jnp.zeros((G, 4, 2, HALF), _F32)
    wxb = wxb.at[:, :, 1, :].set(cols(bW.T, G))
    whh = jnp.zeros((H, 4, 2, HALF), _F32)
    whh = whh.at[0:HALF, :, 0, :].set(cols(fU.T, HALF))
    whh = whh.at[HALF:H, :, 1, :].set(cols(bU.T, HALF))
    bias = jnp.zeros((4, 2, HALF), _F32)
    bias = bias.at[:, 0, :].set((fb1 + fb2).reshape(4, HALF)[perm])
    bias = bias.at[:, 1, :].set((bb1 + bb2).reshape(4, HALF)[perm])
    return (wxf.reshape(G, 4 * H).astype(_BF),
            wxb.reshape(G, 4 * H).astype(_BF),
            whh.reshape(H, 4 * H).astype(_BF),
            bias.reshape(1, 4 * H))


def _head_kernel(trig_ref, h_ref, w1_ref, b1_ref, w2_ref, b2_ref, out_ref,
                 rows_ref):
    for b in range(B):
        rows_ref[pl.ds(b, 1), :] = h_ref[trig_ref[b], pl.ds(b, 1), :]
    z = jnp.tanh(jnp.dot(rows_ref[...].astype(_BF), w1_ref[...],
                         preferred_element_type=_F32) + b1_ref[...])
    out_ref[...] = jnp.dot(z.astype(_BF), w2_ref[...],
                           preferred_element_type=_F32) + b2_ref[...]


def _head(trigger, h_tm, w1, b1, w2, b2):
    return pl.pallas_call(
        _head_kernel,
        in_specs=[
            pl.BlockSpec(memory_space=pltpu.SMEM),
            pl.BlockSpec(memory_space=pltpu.MemorySpace.VMEM),
            pl.BlockSpec(memory_space=pltpu.MemorySpace.VMEM),
            pl.BlockSpec(memory_space=pltpu.MemorySpace.VMEM),
            pl.BlockSpec(memory_space=pltpu.MemorySpace.VMEM),
            pl.BlockSpec(memory_space=pltpu.MemorySpace.VMEM),
        ],
        out_shape=jax.ShapeDtypeStruct((B, 1), _F32),
        scratch_shapes=[pltpu.VMEM((B, H), _F32)],
    )(trigger, h_tm, w1, b1, w2, b2)


def _pack_uni(Wih, Whh, bih, bhh):
    perm = jnp.array([0, 1, 3, 2])
    wih = Wih.T.reshape(IN, 4, H)[:, perm, :].reshape(IN, 4 * H)
    whh = Whh.T.reshape(H, 4, H)[:, perm, :].reshape(H, 4 * H)
    bias = (bih + bhh).reshape(4, H)[perm].reshape(1, 4 * H)
    return wih.astype(_BF), whh.astype(_BF), bias.astype(_F32)


def kernel(x, adj, trigger, emb, ctx_Wih, ctx_Whh, ctx_bih, ctx_bhh, enc0_Wg, enc0_bg, enc0_fWih, enc0_fWhh, enc0_fbih, enc0_fbhh, enc0_bWih, enc0_bWhh, enc0_bbih, enc0_bbhh, enc1_Wg, enc1_bg, enc1_fWih, enc1_fWhh, enc1_fbih, enc1_fbhh, enc1_bWih, enc1_bWhh, enc1_bbih, enc1_bbhh, pre_W1, pre_b1, pre_W2, pre_b2):
    idx_tm = x.T.reshape(BS)
    hx = _row_gather(emb, idx_tm)

    wih, whh, bias = _pack_uni(ctx_Wih, ctx_Whh, ctx_bih, ctx_bhh)
    h = _ctx_lstm(hx, wih, whh, bias)

    p_bm = _perm_tm_to_bm()
    p_tm = _perm_bm_to_tm()
    for Wg, bg, fW, fU, fb1, fb2, bW, bU, bb1, bb2 in (
        (enc0_Wg, enc0_bg, enc0_fWih, enc0_fWhh, enc0_fbih, enc0_fbhh,
         enc0_bWih, enc0_bWhh, enc0_bbih, enc0_bbhh),
        (enc1_Wg, enc1_bg, enc1_fWih, enc1_fWhh, enc1_fbih, enc1_fbhh,
         enc1_bWih, enc1_bWhh, enc1_bbih, enc1_bbhh),
    ):
        h_bm = _row_gather(h.reshape(BS, H), p_bm).reshape(B, S, H)
        g_bm = _gcn(adj, h_bm, Wg.astype(_BF), bg[None, :].astype(_F32))
        g_tm = _row_gather(g_bm.reshape(BS, G), p_tm)
        wxf, wxb, whh_bd, bias_cat = _pack_bilstm(
            fW, fU, fb1, fb2, bW, bU, bb1, bb2)
        h = _bilstm(g_tm, wxf, wxb, whh_bd, bias_cat)

    z = _head(trigger, h, pre_W1.astype(_BF), pre_b1[None, :].astype(_F32),
              pre_W2.astype(_BF), pre_b2[None, :].astype(_F32))
    return z.reshape(B)


# speedup vs baseline: 1.2158x; 1.2158x over previous
import jax
import jax.numpy as jnp
from jax.experimental import pallas as pl
from jax.experimental.pallas import tpu as pltpu
from jax.experimental.pallas import tpu_sc as plsc

B = 16
S = 512
IN = 128
H = 128
G = 128
LH = 256
HALF = 64
BS = B * S
CH = 2
CB = B // CH

_BF = jnp.bfloat16
_F32 = jnp.float32


_NC, _NS = 2, 16
_NW = _NC * _NS
_B_PER_W = BS // _NW


def _row_gather_body(table_hbm, idx_hbm, out_hbm, idx_v, rows_v, sem):
    wid = jax.lax.axis_index("s") * _NC + jax.lax.axis_index("c")
    base = wid * _B_PER_W
    pltpu.sync_copy(idx_hbm.at[pl.ds(base, _B_PER_W)], idx_v)
    pltpu.async_copy(table_hbm.at[idx_v], rows_v, sem).wait()
    pltpu.sync_copy(rows_v, out_hbm.at[pl.ds(base, _B_PER_W)])


def _row_gather(table, idx_flat):
    k = pl.kernel(
        _row_gather_body,
        out_type=jax.ShapeDtypeStruct((BS, table.shape[1]), _F32),
        mesh=plsc.VectorSubcoreMesh(core_axis_name="c", subcore_axis_name="s"),
        scratch_types=[
            pltpu.VMEM((_B_PER_W,), jnp.int32),
            pltpu.VMEM((_B_PER_W, table.shape[1]), _F32),
            pltpu.SemaphoreType.DMA,
        ],
    )
    return k(table, idx_flat)


def _perm_tm_to_bm():
    r = jnp.arange(BS, dtype=jnp.int32)
    return (r % S) * B + r // S


def _perm_bm_to_tm():
    r = jnp.arange(BS, dtype=jnp.int32)
    return (r % B) * S + r // B


def _ctx_lstm_kernel(x_ref, wih_ref, whh_ref, b_ref, out_ref, xp_ref):
    def proj(k, _):
        rows = pl.ds(k * 256, 256)
        xp_ref[rows, :] = jnp.dot(x_ref[rows, :].astype(_BF), wih_ref[...],
                                  preferred_element_type=_F32) + b_ref[...]
        return _
    jax.lax.fori_loop(0, BS // 256, proj, 0)

    def step(t, carry):
        xp_t = xp_ref[pl.ds(t * B, B), :]
        new = []
        for g in range(CH):
            h, c = carry[2 * g], carry[2 * g + 1]
            gates = xp_t[g * CB:(g + 1) * CB, :] + jnp.dot(
                h.astype(_BF), whh_ref[...], preferred_element_type=_F32)
            sig = jax.nn.sigmoid(gates[:, 0:3 * H])
            gg = jnp.tanh(gates[:, 3 * H:4 * H])
            c = sig[:, H:2 * H] * c + sig[:, 0:H] * gg
            h = sig[:, 2 * H:3 * H] * jnp.tanh(c)
            out_ref[t, pl.ds(g * CB, CB), :] = h
            new += [h, c]
        return tuple(new)

    z = jnp.zeros((CB, H), _F32)
    jax.lax.fori_loop(0, S, step, (z, z) * CH, unroll=2)


def _ctx_lstm(x_flat, wih_t, whh_t, bias):
    return pl.pallas_call(
        _ctx_lstm_kernel,
        out_shape=jax.ShapeDtypeStruct((S, B, H), _F32),
        scratch_shapes=[pltpu.VMEM((BS, 4 * H), _F32)],
    )(x_flat, wih_t, whh_t, bias)


def _gcn_kernel(adj_ref, h_ref, wg_ref, bg_ref, out_ref):
    adj = adj_ref[0]
    rs = jnp.sum(adj, axis=1, keepdims=True) + 1e-8
    m = jnp.dot(adj.astype(_BF), h_ref[0].astype(_BF),
                preferred_element_type=_F32)
    m = m / rs
    g = jnp.dot(m.astype(_BF), wg_ref[...], preferred_element_type=_F32)
    out_ref[0] = jnp.maximum(g + bg_ref[...], 0.0)


def _gcn(adj, h_bm, wg_t, bg):
    return pl.pallas_call(
        _gcn_kernel,
        grid=(B,),
        in_specs=[
            pl.BlockSpec((1, S, S), lambda b: (b, 0, 0)),
            pl.BlockSpec((1, S, H), lambda b: (b, 0, 0)),
            pl.BlockSpec((H, G), lambda b: (0, 0)),
            pl.BlockSpec((1, G), lambda b: (0, 0)),
        ],
        out_specs=pl.BlockSpec((1, S, G), lambda b: (b, 0, 0)),
        out_shape=jax.ShapeDtypeStruct((B, S, G), _F32),
    )(adj, h_bm, wg_t, bg)


def _bilstm_kernel(x_ref, wxf_ref, wxb_ref, whh_ref, b_ref, out_ref,
                   xpf_ref, xpb_ref):
    def proj(k, _):
        rows = pl.ds(k * 256, 256)
        xb = x_ref[rows, :].astype(_BF)
        xpf_ref[rows, :] = (jnp.dot(xb, wxf_ref[...],
                                    preferred_element_type=_F32)
                            + b_ref[...]).astype(_BF)
        xpb_ref[rows, :] = jnp.dot(xb, wxb_ref[...],
                                   preferred_element_type=_F32).astype(_BF)
        return _
    jax.lax.fori_loop(0, BS // 256, proj, 0)

    def step(t, carry):
        tb = S - 1 - t
        xpf_t = xpf_ref[pl.ds(t * B, B), :].astype(_F32)
        xpb_t = xpb_ref[pl.ds(tb * B, B), :].astype(_F32)
        new = []
        for g in range(CH):
            h, c = carry[2 * g], carry[2 * g + 1]
            rows = slice(g * CB, (g + 1) * CB)
            gates = xpf_t[rows, :] + xpb_t[rows, :] + jnp.dot(
                h.astype(_BF), whh_ref[...], preferred_element_type=_F32)
            sig = jax.nn.sigmoid(gates[:, 0:3 * H])
            gg = jnp.tanh(gates[:, 3 * H:4 * H])
            c = sig[:, H:2 * H] * c + sig[:, 0:H] * gg
            h = sig[:, 2 * H:3 * H] * jnp.tanh(c)
            out_ref[t, pl.ds(g * CB, CB), 0:HALF] = h[:, 0:HALF]
            out_ref[tb, pl.ds(g * CB, CB), HALF:H] = h[:, HALF:H]
            new += [h, c]
        return tuple(new)

    z = jnp.zeros((CB, H), _F32)
    jax.lax.fori_loop(0, S, step, (z, z) * CH, unroll=2)


def _bilstm(x_flat, wxf, wxb, whh_bd, bias_cat):
    return pl.pallas_call(
        _bilstm_kernel,
        out_shape=jax.ShapeDtypeStruct((S, B, H), _F32),
        scratch_shapes=[
            pltpu.VMEM((BS, 4 * H), _BF),
            pltpu.VMEM((BS, 4 * H), _BF),
        ],
    )(x_flat, wxf, wxb, whh_bd, bias_cat)


def _pack_bilstm(fW, fU, fb1, fb2, bW, bU, bb1, bb2):
    perm = jnp.array([0, 1, 3, 2])

    def cols(Wt, n_in):
        return Wt.reshape(n_in, 4, HALF)[:, perm, :]

    wxf = jnp.zeros((G, 4, 2, HALF), _F32)
    wxf = wxf.at[:, :, 0, :].set(cols(fW.T, G))
    wxb = jnp.zeros((G, 4, 2, HALF), _F32)
    wxb = wxb.at[:, :, 1, :].set(cols(bW.T, G))
    whh = jnp.zeros((H, 4, 2, HALF), _F32)
    whh = whh.at[0:HALF, :, 0, :].set(cols(fU.T, HALF))
    whh = whh.at[HALF:H, :, 1, :].set(cols(bU.T, HALF))
    bias = jnp.zeros((4, 2, HALF), _F32)
    bias = bias.at[:, 0, :].set((fb1 + fb2).reshape(4, HALF)[perm])
    bias = bias.at[:, 1, :].set((bb1 + bb2).reshape(4, HALF)[perm])
    return (wxf.reshape(G, 4 * H).astype(_BF),
            wxb.reshape(G, 4 * H).astype(_BF),
            whh.reshape(H, 4 * H).astype(_BF),
            bias.reshape(1, 4 * H))


def _head_kernel(trig_ref, h_ref, w1_ref, b1_ref, w2_ref, b2_ref, out_ref,
                 rows_ref):
    for b in range(B):
        rows_ref[pl.ds(b, 1), :] = h_ref[trig_ref[b], pl.ds(b, 1), :]
    z = jnp.tanh(jnp.dot(rows_ref[...].astype(_BF), w1_ref[...],
                         preferred_element_type=_F32) + b1_ref[...])
    out_ref[...] = jnp.dot(z.astype(_BF), w2_ref[...],
                           preferred_element_type=_F32) + b2_ref[...]


def _head(trigger, h_tm, w1, b1, w2, b2):
    return pl.pallas_call(
        _head_kernel,
        in_specs=[
            pl.BlockSpec(memory_space=pltpu.SMEM),
            pl.BlockSpec(memory_space=pltpu.MemorySpace.VMEM),
            pl.BlockSpec(memory_space=pltpu.MemorySpace.VMEM),
            pl.BlockSpec(memory_space=pltpu.MemorySpace.VMEM),
            pl.BlockSpec(memory_space=pltpu.MemorySpace.VMEM),
            pl.BlockSpec(memory_space=pltpu.MemorySpace.VMEM),
        ],
        out_shape=jax.ShapeDtypeStruct((B, 1), _F32),
        scratch_shapes=[pltpu.VMEM((B, H), _F32)],
    )(trigger, h_tm, w1, b1, w2, b2)


def _pack_uni(Wih, Whh, bih, bhh):
    perm = jnp.array([0, 1, 3, 2])
    wih = Wih.T.reshape(IN, 4, H)[:, perm, :].reshape(IN, 4 * H)
    whh = Whh.T.reshape(H, 4, H)[:, perm, :].reshape(H, 4 * H)
    bias = (bih + bhh).reshape(4, H)[perm].reshape(1, 4 * H)
    return wih.astype(_BF), whh.astype(_BF), bias.astype(_F32)


def kernel(x, adj, trigger, emb, ctx_Wih, ctx_Whh, ctx_bih, ctx_bhh, enc0_Wg, enc0_bg, enc0_fWih, enc0_fWhh, enc0_fbih, enc0_fbhh, enc0_bWih, enc0_bWhh, enc0_bbih, enc0_bbhh, enc1_Wg, enc1_bg, enc1_fWih, enc1_fWhh, enc1_fbih, enc1_fbhh, enc1_bWih, enc1_bWhh, enc1_bbih, enc1_bbhh, pre_W1, pre_b1, pre_W2, pre_b2):
    idx_tm = x.T.reshape(BS)
    hx = _row_gather(emb, idx_tm)

    wih, whh, bias = _pack_uni(ctx_Wih, ctx_Whh, ctx_bih, ctx_bhh)
    h = _ctx_lstm(hx, wih, whh, bias)

    p_bm = _perm_tm_to_bm()
    p_tm = _perm_bm_to_tm()
    for Wg, bg, fW, fU, fb1, fb2, bW, bU, bb1, bb2 in (
        (enc0_Wg, enc0_bg, enc0_fWih, enc0_fWhh, enc0_fbih, enc0_fbhh,
         enc0_bWih, enc0_bWhh, enc0_bbih, enc0_bbhh),
        (enc1_Wg, enc1_bg, enc1_fWih, enc1_fWhh, enc1_fbih, enc1_fbhh,
         enc1_bWih, enc1_bWhh, enc1_bbih, enc1_bbhh),
    ):
        h_bm = _row_gather(h.reshape(BS, H), p_bm).reshape(B, S, H)
        g_bm = _gcn(adj, h_bm, Wg.astype(_BF), bg[None, :].astype(_F32))
        g_tm = _row_gather(g_bm.reshape(BS, G), p_tm)
        wxf, wxb, whh_bd, bias_cat = _pack_bilstm(
            fW, fU, fb1, fb2, bW, bU, bb1, bb2)
        h = _bilstm(g_tm, wxf, wxb, whh_bd, bias_cat)

    z = _head(trigger, h, pre_W1.astype(_BF), pre_b1[None, :].astype(_F32),
              pre_W2.astype(_BF), pre_b2[None, :].astype(_F32))
    return z.reshape(B)


# speedup vs baseline: 1.2608x; 1.0370x over previous
import jax
import jax.numpy as jnp
from jax.experimental import pallas as pl
from jax.experimental.pallas import tpu as pltpu
from jax.experimental.pallas import tpu_sc as plsc

B = 16
S = 512
IN = 128
H = 128
G = 128
LH = 256
HALF = 64
BS = B * S
CH = 2
CB = B // CH

_BF = jnp.bfloat16
_F32 = jnp.float32


_NC, _NS = 2, 16
_NW = _NC * _NS
_B_PER_W = BS // _NW


def _row_gather_body(table_hbm, idx_hbm, out_hbm, idx_v, rows_v, sem):
    wid = jax.lax.axis_index("s") * _NC + jax.lax.axis_index("c")
    base = wid * _B_PER_W
    pltpu.sync_copy(idx_hbm.at[pl.ds(base, _B_PER_W)], idx_v)
    pltpu.async_copy(table_hbm.at[idx_v], rows_v, sem).wait()
    pltpu.sync_copy(rows_v, out_hbm.at[pl.ds(base, _B_PER_W)])


def _row_gather(table, idx_flat):
    k = pl.kernel(
        _row_gather_body,
        out_type=jax.ShapeDtypeStruct((BS, table.shape[1]), _F32),
        mesh=plsc.VectorSubcoreMesh(core_axis_name="c", subcore_axis_name="s"),
        scratch_types=[
            pltpu.VMEM((_B_PER_W,), jnp.int32),
            pltpu.VMEM((_B_PER_W, table.shape[1]), _F32),
            pltpu.SemaphoreType.DMA,
        ],
    )
    return k(table, idx_flat)


def _perm_tm_to_bm():
    r = jnp.arange(BS, dtype=jnp.int32)
    return (r % S) * B + r // S


def _perm_bm_to_tm():
    r = jnp.arange(BS, dtype=jnp.int32)
    return (r % B) * S + r // B


def _ctx_lstm_kernel(x_ref, wih_ref, whh_ref, b_ref, out_ref, xp_ref):
    def proj(k, _):
        rows = pl.ds(k * 256, 256)
        xp_ref[rows, :] = jnp.dot(x_ref[rows, :].astype(_BF), wih_ref[...],
                                  preferred_element_type=_F32) + b_ref[...]
        return _
    jax.lax.fori_loop(0, BS // 256, proj, 0)

    def step(t, carry):
        xp_t = xp_ref[pl.ds(t * B, B), :]
        new = []
        for g in range(CH):
            h, c = carry[2 * g], carry[2 * g + 1]
            gates = xp_t[g * CB:(g + 1) * CB, :] + jnp.dot(
                h.astype(_BF), whh_ref[...], preferred_element_type=_F32)
            sig = jax.nn.sigmoid(gates[:, 0:3 * H])
            gg = jnp.tanh(gates[:, 3 * H:4 * H])
            c = sig[:, H:2 * H] * c + sig[:, 0:H] * gg
            h = sig[:, 2 * H:3 * H] * jnp.tanh(c)
            out_ref[t, pl.ds(g * CB, CB), :] = h
            new += [h, c]
        return tuple(new)

    z = jnp.zeros((CB, H), _F32)
    jax.lax.fori_loop(0, S, step, (z, z) * CH, unroll=4)


def _ctx_lstm(x_flat, wih_t, whh_t, bias):
    return pl.pallas_call(
        _ctx_lstm_kernel,
        out_shape=jax.ShapeDtypeStruct((S, B, H), _F32),
        scratch_shapes=[pltpu.VMEM((BS, 4 * H), _F32)],
    )(x_flat, wih_t, whh_t, bias)


def _gcn_kernel(adj_ref, h_ref, wg_ref, bg_ref, out_ref):
    adj = adj_ref[0]
    rs = jnp.sum(adj, axis=1, keepdims=True) + 1e-8
    m = jnp.dot(adj.astype(_BF), h_ref[0].astype(_BF),
                preferred_element_type=_F32)
    m = m / rs
    g = jnp.dot(m.astype(_BF), wg_ref[...], preferred_element_type=_F32)
    out_ref[0] = jnp.maximum(g + bg_ref[...], 0.0)


def _gcn(adj, h_bm, wg_t, bg):
    return pl.pallas_call(
        _gcn_kernel,
        grid=(B,),
        in_specs=[
            pl.BlockSpec((1, S, S), lambda b: (b, 0, 0)),
            pl.BlockSpec((1, S, H), lambda b: (b, 0, 0)),
            pl.BlockSpec((H, G), lambda b: (0, 0)),
            pl.BlockSpec((1, G), lambda b: (0, 0)),
        ],
        out_specs=pl.BlockSpec((1, S, G), lambda b: (b, 0, 0)),
        out_shape=jax.ShapeDtypeStruct((B, S, G), _F32),
    )(adj, h_bm, wg_t, bg)


def _bilstm_kernel(x_ref, wxf_ref, wxb_ref, whh_ref, b_ref, out_ref,
                   xpf_ref, xpb_ref):
    def proj(k, _):
        rows = pl.ds(k * 256, 256)
        xb = x_ref[rows, :].astype(_BF)
        xpf_ref[rows, :] = (jnp.dot(xb, wxf_ref[...],
                                    preferred_element_type=_F32)
                            + b_ref[...]).astype(_BF)
        xpb_ref[rows, :] = jnp.dot(xb, wxb_ref[...],
                                   preferred_element_type=_F32).astype(_BF)
        return _
    jax.lax.fori_loop(0, BS // 256, proj, 0)

    def step(t, carry):
        tb = S - 1 - t
        xpf_t = xpf_ref[pl.ds(t * B, B), :].astype(_F32)
        xpb_t = xpb_ref[pl.ds(tb * B, B), :].astype(_F32)
        new = []
        for g in range(CH):
            h, c = carry[2 * g], carry[2 * g + 1]
            rows = slice(g * CB, (g + 1) * CB)
            gates = xpf_t[rows, :] + xpb_t[rows, :] + jnp.dot(
                h.astype(_BF), whh_ref[...], preferred_element_type=_F32)
            sig = jax.nn.sigmoid(gates[:, 0:3 * H])
            gg = jnp.tanh(gates[:, 3 * H:4 * H])
            c = sig[:, H:2 * H] * c + sig[:, 0:H] * gg
            h = sig[:, 2 * H:3 * H] * jnp.tanh(c)
            out_ref[t, pl.ds(g * CB, CB), 0:HALF] = h[:, 0:HALF]
            out_ref[tb, pl.ds(g * CB, CB), HALF:H] = h[:, HALF:H]
            new += [h, c]
        return tuple(new)

    z = jnp.zeros((CB, H), _F32)
    jax.lax.fori_loop(0, S, step, (z, z) * CH, unroll=4)


def _bilstm(x_flat, wxf, wxb, whh_bd, bias_cat):
    return pl.pallas_call(
        _bilstm_kernel,
        out_shape=jax.ShapeDtypeStruct((S, B, H), _F32),
        scratch_shapes=[
            pltpu.VMEM((BS, 4 * H), _BF),
            pltpu.VMEM((BS, 4 * H), _BF),
        ],
    )(x_flat, wxf, wxb, whh_bd, bias_cat)


def _pack_bilstm(fW, fU, fb1, fb2, bW, bU, bb1, bb2):
    perm = jnp.array([0, 1, 3, 2])

    def cols(Wt, n_in):
        return Wt.reshape(n_in, 4, HALF)[:, perm, :]

    wxf = jnp.zeros((G, 4, 2, HALF), _F32)
    wxf = wxf.at[:, :, 0, :].set(cols(fW.T, G))
    wxb = jnp.zeros((G, 4, 2, HALF), _F32)
    wxb = wxb.at[:, :, 1, :].set(cols(bW.T, G))
    whh = jnp.zeros((H, 4, 2, HALF), _F32)
    whh = whh.at[0:HALF, :, 0, :].set(cols(fU.T, HALF))
    whh = whh.at[HALF:H, :, 1, :].set(cols(bU.T, HALF))
    bias = jnp.zeros((4, 2, HALF), _F32)
    bias = bias.at[:, 0, :].set((fb1 + fb2).reshape(4, HALF)[perm])
    bias = bias.at[:, 1, :].set((bb1 + bb2).reshape(4, HALF)[perm])
    return (wxf.reshape(G, 4 * H).astype(_BF),
            wxb.reshape(G, 4 * H).astype(_BF),
            whh.reshape(H, 4 * H).astype(_BF),
            bias.reshape(1, 4 * H))


def _head_kernel(trig_ref, h_ref, w1_ref, b1_ref, w2_ref, b2_ref, out_ref,
                 rows_ref):
    for b in range(B):
        rows_ref[pl.ds(b, 1), :] = h_ref[trig_ref[b], pl.ds(b, 1), :]
    z = jnp.tanh(jnp.dot(rows_ref[...].astype(_BF), w1_ref[...],
                         preferred_element_type=_F32) + b1_ref[...])
    out_ref[...] = jnp.dot(z.astype(_BF), w2_ref[...],
                           preferred_element_type=_F32) + b2_ref[...]


def _head(trigger, h_tm, w1, b1, w2, b2):
    return pl.pallas_call(
        _head_kernel,
        in_specs=[
            pl.BlockSpec(memory_space=pltpu.SMEM),
            pl.BlockSpec(memory_space=pltpu.MemorySpace.VMEM),
            pl.BlockSpec(memory_space=pltpu.MemorySpace.VMEM),
            pl.BlockSpec(memory_space=pltpu.MemorySpace.VMEM),
            pl.BlockSpec(memory_space=pltpu.MemorySpace.VMEM),
            pl.BlockSpec(memory_space=pltpu.MemorySpace.VMEM),
        ],
        out_shape=jax.ShapeDtypeStruct((B, 1), _F32),
        scratch_shapes=[pltpu.VMEM((B, H), _F32)],
    )(trigger, h_tm, w1, b1, w2, b2)


def _pack_uni(Wih, Whh, bih, bhh):
    perm = jnp.array([0, 1, 3, 2])
    wih = Wih.T.reshape(IN, 4, H)[:, perm, :].reshape(IN, 4 * H)
    whh = Whh.T.reshape(H, 4, H)[:, perm, :].reshape(H, 4 * H)
    bias = (bih + bhh).reshape(4, H)[perm].reshape(1, 4 * H)
    return wih.astype(_BF), whh.astype(_BF), bias.astype(_F32)


def kernel(x, adj, trigger, emb, ctx_Wih, ctx_Whh, ctx_bih, ctx_bhh, enc0_Wg, enc0_bg, enc0_fWih, enc0_fWhh, enc0_fbih, enc0_fbhh, enc0_bWih, enc0_bWhh, enc0_bbih, enc0_bbhh, enc1_Wg, enc1_bg, enc1_fWih, enc1_fWhh, enc1_fbih, enc1_fbhh, enc1_bWih, enc1_bWhh, enc1_bbih, enc1_bbhh, pre_W1, pre_b1, pre_W2, pre_b2):
    idx_tm = x.T.reshape(BS)
    hx = _row_gather(emb, idx_tm)

    wih, whh, bias = _pack_uni(ctx_Wih, ctx_Whh, ctx_bih, ctx_bhh)
    h = _ctx_lstm(hx, wih, whh, bias)

    p_bm = _perm_tm_to_bm()
    p_tm = _perm_bm_to_tm()
    for Wg, bg, fW, fU, fb1, fb2, bW, bU, bb1, bb2 in (
        (enc0_Wg, enc0_bg, enc0_fWih, enc0_fWhh, enc0_fbih, enc0_fbhh,
         enc0_bWih, enc0_bWhh, enc0_bbih, enc0_bbhh),
        (enc1_Wg, enc1_bg, enc1_fWih, enc1_fWhh, enc1_fbih, enc1_fbhh,
         enc1_bWih, enc1_bWhh, enc1_bbih, enc1_bbhh),
    ):
        h_bm = _row_gather(h.reshape(BS, H), p_bm).reshape(B, S, H)
        g_bm = _gcn(adj, h_bm, Wg.astype(_BF), bg[None, :].astype(_F32))
        g_tm = _row_gather(g_bm.reshape(BS, G), p_tm)
        wxf, wxb, whh_bd, bias_cat = _pack_bilstm(
            fW, fU, fb1, fb2, bW, bU, bb1, bb2)
        h = _bilstm(g_tm, wxf, wxb, whh_bd, bias_cat)

    z = _head(trigger, h, pre_W1.astype(_BF), pre_b1[None, :].astype(_F32),
              pre_W2.astype(_BF), pre_b2[None, :].astype(_F32))
    return z.reshape(B)
